# baseline (speedup 1.0000x reference)
import functools

import jax
import jax.numpy as jnp
from jax import lax
from jax.experimental import pallas as pl
from jax.experimental.pallas import tpu as pltpu
from jax.experimental.pallas import tpu_sc as plsc

L = 16
NC = 2
NS = 16
NW = NC * NS

B = 16384
D = 32
CHUNK = B // NW
NSTREAM = 4
IDXW = CHUNK // NSTREAM


def _mf_body(users_hbm, items_hbm, user_table_hbm, item_table_hbm, out_hbm,
             uidx_v, iidx_v, urows_v, irows_v, out_v, sem):
    wid = lax.axis_index("s") * NC + lax.axis_index("c")
    base = wid * CHUNK

    pltpu.sync_copy(users_hbm.at[wid], uidx_v)
    pltpu.sync_copy(items_hbm.at[wid], iidx_v)

    copies = []
    for j in range(NSTREAM):
        copies.append(pltpu.async_copy(
            user_table_hbm.at[uidx_v.at[j]],
            urows_v.at[pl.ds(j * IDXW, IDXW)], sem))
        copies.append(pltpu.async_copy(
            item_table_hbm.at[iidx_v.at[j]],
            irows_v.at[pl.ds(j * IDXW, IDXW)], sem))
    for c in copies:
        c.wait()

    def group(g, carry):
        row = g * L + lax.iota(jnp.int32, L)
        acc = jnp.zeros((L,), jnp.float32)
        for d in range(D):
            col = jnp.full((L,), d, jnp.int32)
            u = plsc.load_gather(urows_v, [row, col])
            v = plsc.load_gather(irows_v, [row, col])
            acc = acc + u * v
        out_v[pl.ds(g * L, L)] = acc
        return carry

    lax.fori_loop(0, CHUNK // L, group, 0)

    pltpu.sync_copy(out_v, out_hbm.at[pl.ds(base, CHUNK)])


@functools.partial(
    pl.kernel,
    out_type=jax.ShapeDtypeStruct((B,), jnp.float32),
    mesh=plsc.VectorSubcoreMesh(core_axis_name="c", subcore_axis_name="s"),
    scratch_types=[
        pltpu.VMEM((NSTREAM, IDXW), jnp.int32),
        pltpu.VMEM((NSTREAM, IDXW), jnp.int32),
        pltpu.VMEM((CHUNK, D), jnp.float32),
        pltpu.VMEM((CHUNK, D), jnp.float32),
        pltpu.VMEM((CHUNK,), jnp.float32),
        pltpu.SemaphoreType.DMA,
    ],
    compiler_params=pltpu.CompilerParams(
        needs_layout_passes=False, use_tc_tiling_on_sc=False),
)
def _mf(users_hbm, items_hbm, user_table_hbm, item_table_hbm, out_hbm,
        uidx_v, iidx_v, urows_v, irows_v, out_v, sem):
    _mf_body(users_hbm, items_hbm, user_table_hbm, item_table_hbm, out_hbm,
             uidx_v, iidx_v, urows_v, irows_v, out_v, sem)


def kernel(users, items, user_table, item_table):
    u = users.astype(jnp.int32).reshape(NW, NSTREAM, IDXW)
    it = items.astype(jnp.int32).reshape(NW, NSTREAM, IDXW)
    return _mf(u, it, user_table, item_table)
